# baseline (speedup 1.0000x reference)
import jax
import jax.numpy as jnp
from jax.experimental import pallas as pl

_NUM_EXPERTS = 64
_INV_TEMP = 0.5


def _router_block(x_ref, w_ref, b_ref, o_ref):
    logits = jax.lax.dot_general(
        x_ref[...], w_ref[...],
        dimension_numbers=(((1,), (1,)), ((), ())),
        preferred_element_type=jnp.float32,
    )
    logits = (logits + b_ref[...]) * _INV_TEMP
    m = jnp.max(logits, axis=-1, keepdims=True)
    e = jnp.exp(logits - m)
    o_ref[...] = e * (1.0 / jnp.sum(e, axis=-1, keepdims=True))


def kernel(x, gate_w, gate_b):
    n_tokens, d = x.shape
    e = gate_w.shape[0]
    bt = 1024
    b2d = gate_b.reshape(1, e)
    return pl.pallas_call(
        _router_block,
        grid=(n_tokens // bt,),
        in_specs=[
            pl.BlockSpec((bt, d), lambda i: (i, 0)),
            pl.BlockSpec((e, d), lambda i: (0, 0)),
            pl.BlockSpec((1, e), lambda i: (0, 0)),
        ],
        out_specs=pl.BlockSpec((bt, e), lambda i: (i, 0)),
        out_shape=jax.ShapeDtypeStruct((n_tokens, e), jnp.float32),
    )(x, gate_w, b2d)


# speedup vs baseline: 1.0132x; 1.0132x over previous
import jax
import jax.numpy as jnp
from jax.experimental import pallas as pl

_NUM_EXPERTS = 64
_INV_TEMP = 0.5


def _router_block(x_ref, w_ref, b_ref, o_ref):
    logits = jax.lax.dot_general(
        x_ref[...].astype(jnp.bfloat16), w_ref[...].astype(jnp.bfloat16),
        dimension_numbers=(((1,), (1,)), ((), ())),
        preferred_element_type=jnp.float32,
    )
    logits = (logits + b_ref[...]) * _INV_TEMP
    m = jnp.max(logits, axis=-1, keepdims=True)
    e = jnp.exp(logits - m)
    o_ref[...] = e * (1.0 / jnp.sum(e, axis=-1, keepdims=True))


def kernel(x, gate_w, gate_b):
    n_tokens, d = x.shape
    e = gate_w.shape[0]
    bt = 1024
    b2d = gate_b.reshape(1, e)
    return pl.pallas_call(
        _router_block,
        grid=(n_tokens // bt,),
        in_specs=[
            pl.BlockSpec((bt, d), lambda i: (i, 0)),
            pl.BlockSpec((e, d), lambda i: (0, 0)),
            pl.BlockSpec((1, e), lambda i: (0, 0)),
        ],
        out_specs=pl.BlockSpec((bt, e), lambda i: (i, 0)),
        out_shape=jax.ShapeDtypeStruct((n_tokens, e), jnp.float32),
    )(x, gate_w, b2d)


# speedup vs baseline: 1.0207x; 1.0074x over previous
import jax
import jax.numpy as jnp
from jax.experimental import pallas as pl

_INV_TEMP = 0.5
_BT = 1024
_SPLIT = 4


def _router_block(*refs):
    x_refs = refs[:_SPLIT]
    w_ref, b_ref, o_ref = refs[_SPLIT:]
    w = w_ref[...].astype(jnp.bfloat16)
    sub = _BT // _SPLIT
    for q in range(_SPLIT):
        logits = jax.lax.dot_general(
            x_refs[q][...].astype(jnp.bfloat16), w,
            dimension_numbers=(((1,), (1,)), ((), ())),
            preferred_element_type=jnp.float32,
        )
        logits = (logits + b_ref[...]) * _INV_TEMP
        m = jnp.max(logits, axis=-1, keepdims=True)
        e = jnp.exp(logits - m)
        o_ref[q * sub:(q + 1) * sub, :] = e * (
            1.0 / jnp.sum(e, axis=-1, keepdims=True))


def kernel(x, gate_w, gate_b):
    n_tokens, d = x.shape
    ne = gate_w.shape[0]
    sub = _BT // _SPLIT
    b2d = gate_b.reshape(1, ne)

    def _mk_spec(q):
        return pl.BlockSpec((sub, d), lambda i, q=q: (i * _SPLIT + q, 0))

    return pl.pallas_call(
        _router_block,
        grid=(n_tokens // _BT,),
        in_specs=[_mk_spec(q) for q in range(_SPLIT)] + [
            pl.BlockSpec((ne, d), lambda i: (0, 0)),
            pl.BlockSpec((1, ne), lambda i: (0, 0)),
        ],
        out_specs=pl.BlockSpec((_BT, ne), lambda i: (i, 0)),
        out_shape=jax.ShapeDtypeStruct((n_tokens, ne), jnp.float32),
    )(*([x] * _SPLIT), gate_w, b2d)
